# baseline (speedup 1.0000x reference)
import jax
import jax.numpy as jnp
from jax import lax
from jax.experimental import pallas as pl
from jax.experimental.pallas import tpu as pltpu

B, C, T, Q, P = 4096, 4096, 32, 64, 128
RB = 128


def _tc_body(x_ref, o_ref):
    v = x_ref[...]
    idx = P - 1 - lax.broadcasted_iota(jnp.int32, (RB, 8, P), 2)
    v = jnp.take_along_axis(v, idx, axis=2)
    j = lax.broadcasted_iota(jnp.int32, (RB, 8, P), 1)
    sidx = 6 - j + 2 * (j & 1)
    o_ref[...] = jnp.take_along_axis(v, sidx, axis=1)


@jax.jit
def _flip(xv):
    return pl.pallas_call(
        _tc_body,
        grid=(B // RB, Q // 8),
        in_specs=[pl.BlockSpec((RB, 8, P), lambda i, u: (i, Q // 8 - 1 - u, 0))],
        out_specs=pl.BlockSpec((RB, 8, P), lambda i, u: (i, u, 0)),
        out_shape=jax.ShapeDtypeStruct((B, Q, P), jnp.float32),
    )(xv)


def kernel(x, c):
    xv = x.reshape(B, T, P, 2).transpose(0, 1, 3, 2).reshape(B, Q, P)
    yv = _flip(xv)
    return yv.reshape(B, T, 2, P).transpose(0, 1, 3, 2).reshape(B, C, 2)
